# baseline (speedup 1.0000x reference)
import functools

import jax
import jax.numpy as jnp
from jax import lax
from jax.experimental import pallas as pl
from jax.experimental.pallas import tpu as pltpu
from jax.experimental.pallas import tpu_sc as plsc

B = 4096
L = 200
C = 100001
T1 = 12
BASE = 16
N = B * L

_info = plsc.get_sparse_core_info()
NC, NS, LANES = _info.num_cores, _info.num_subcores, _info.num_lanes
NW = NC * NS
TOK_PER_W = N // NW
CHUNK = 1024
NCHUNK = TOK_PER_W // CHUNK


@functools.partial(
    pl.kernel,
    mesh=plsc.VectorSubcoreMesh(core_axis_name="c", subcore_axis_name="s"),
    out_type=jax.ShapeDtypeStruct((2 * N, BASE), jnp.float32),
    compiler_params=pltpu.CompilerParams(
        needs_layout_passes=False, use_tc_tiling_on_sc=False
    ),
    scratch_types=[
        pltpu.VMEM((CHUNK,), jnp.int32),
        pltpu.VMEM((CHUNK,), jnp.int32),
        pltpu.VMEM((CHUNK,), jnp.int32),
        pltpu.VMEM((2 * CHUNK,), jnp.int32),
        pltpu.VMEM((2 * CHUNK, BASE), jnp.float32),
        pltpu.SemaphoreType.DMA,
    ],
)
def _sc_gather(table_hbm, t1_hbm, t2_hbm, item_hbm, out_hbm,
               t1_v, t2_v, item_v, idx_v, rows_v, sem):
    wid = lax.axis_index("s") * NC + lax.axis_index("c")
    base = wid * TOK_PER_W
    lane = lax.iota(jnp.int32, LANES)

    def chunk_body(k, carry):
        off = base + k * CHUNK
        pltpu.sync_copy(t1_hbm.at[pl.ds(off, CHUNK)], t1_v)
        pltpu.sync_copy(t2_hbm.at[pl.ds(off, CHUNK)], t2_v)
        pltpu.sync_copy(item_hbm.at[pl.ds(off, CHUNK)], item_v)

        def vec_body(j, carry2):
            sl = pl.ds(j * LANES, LANES)
            it = item_v[sl]
            m_idx = t1_v[sl] * C + it
            w_idx = (t2_v[sl] + T1) * C + it
            pos = lane * 2 + j * (2 * LANES)
            plsc.store_scatter(idx_v, [pos], m_idx)
            plsc.store_scatter(idx_v, [pos + 1], w_idx)
            return carry2

        lax.fori_loop(0, CHUNK // LANES, vec_body, 0)
        pltpu.async_copy(table_hbm.at[idx_v], rows_v, sem).wait()
        pltpu.sync_copy(rows_v, out_hbm.at[pl.ds(2 * off, 2 * CHUNK)])
        return carry

    lax.fori_loop(0, NCHUNK, chunk_body, 0)


def kernel(log_seqs, time1_seqs, time2_seqs, month_pop_table, week_pop_table):
    mt = month_pop_table.reshape(T1, BASE, C).transpose(0, 2, 1).reshape(T1 * C, BASE)
    wt = week_pop_table.reshape(-1, BASE, C).transpose(0, 2, 1).reshape(-1, BASE)
    table = jnp.concatenate([mt, wt], axis=0)
    t1 = time1_seqs.reshape(-1).astype(jnp.int32)
    t2 = time2_seqs.reshape(-1).astype(jnp.int32)
    item = log_seqs.reshape(-1).astype(jnp.int32)
    rows = _sc_gather(table, t1, t2, item)
    return rows.reshape(B, L, 2 * BASE)


# speedup vs baseline: 3.3067x; 3.3067x over previous
import functools

import jax
import jax.numpy as jnp
from jax import lax
from jax.experimental import pallas as pl
from jax.experimental.pallas import tpu as pltpu
from jax.experimental.pallas import tpu_sc as plsc

B = 4096
L = 200
C = 100001
T1 = 12
BASE = 16
N = B * L

_info = plsc.get_sparse_core_info()
NC, NS, LANES = _info.num_cores, _info.num_subcores, _info.num_lanes
NW = NC * NS
TOK_PER_W = N // NW
CHUNK = 1024
NCHUNK = TOK_PER_W // CHUNK


@functools.partial(
    pl.kernel,
    mesh=plsc.VectorSubcoreMesh(core_axis_name="c", subcore_axis_name="s"),
    out_type=jax.ShapeDtypeStruct((2 * N, BASE), jnp.float32),
    compiler_params=pltpu.CompilerParams(
        needs_layout_passes=False, use_tc_tiling_on_sc=False
    ),
    scratch_types=[
        pltpu.VMEM((CHUNK,), jnp.int32),
        pltpu.VMEM((CHUNK,), jnp.int32),
        pltpu.VMEM((CHUNK,), jnp.int32),
        pltpu.VMEM((2 * CHUNK,), jnp.int32),
        pltpu.VMEM((2 * CHUNK, BASE), jnp.float32),
        pltpu.SemaphoreType.DMA,
    ],
)
def _sc_gather(table_hbm, t1_hbm, t2_hbm, item_hbm, out_hbm,
               t1_v, t2_v, item_v, idx_v, rows_v, sem):
    wid = lax.axis_index("s") * NC + lax.axis_index("c")
    base = wid * TOK_PER_W
    lane = lax.iota(jnp.int32, LANES)

    def chunk_body(k, carry):
        off = base + k * CHUNK
        pltpu.sync_copy(t1_hbm.at[pl.ds(off, CHUNK)], t1_v)
        pltpu.sync_copy(t2_hbm.at[pl.ds(off, CHUNK)], t2_v)
        pltpu.sync_copy(item_hbm.at[pl.ds(off, CHUNK)], item_v)

        def vec_body(j, carry2):
            sl = pl.ds(j * LANES, LANES)
            it = item_v[sl] * 17
            m_idx = it + t1_v[sl]
            w_idx = it + (t2_v[sl] + T1)
            pos = lane * 2 + j * (2 * LANES)
            plsc.store_scatter(idx_v, [pos], m_idx)
            plsc.store_scatter(idx_v, [pos + 1], w_idx)
            return carry2

        lax.fori_loop(0, CHUNK // LANES, vec_body, 0)
        pltpu.async_copy(table_hbm.at[idx_v], rows_v, sem).wait()
        pltpu.sync_copy(rows_v, out_hbm.at[pl.ds(2 * off, 2 * CHUNK)])
        return carry

    lax.fori_loop(0, NCHUNK, chunk_body, 0)


def kernel(log_seqs, time1_seqs, time2_seqs, month_pop_table, week_pop_table):
    table = (
        jnp.concatenate([month_pop_table, week_pop_table], axis=0)
        .T.reshape(17 * C, BASE)
    )
    t1 = time1_seqs.reshape(-1).astype(jnp.int32)
    t2 = time2_seqs.reshape(-1).astype(jnp.int32)
    item = log_seqs.reshape(-1).astype(jnp.int32)
    rows = _sc_gather(table, t1, t2, item)
    return rows.reshape(B, L, 2 * BASE)


# speedup vs baseline: 4.3460x; 1.3143x over previous
import functools

import jax
import jax.numpy as jnp
from jax import lax
from jax.experimental import pallas as pl
from jax.experimental.pallas import tpu as pltpu
from jax.experimental.pallas import tpu_sc as plsc

B = 4096
L = 200
C = 100001
T1 = 12
BASE = 16
N = B * L

_info = plsc.get_sparse_core_info()
NC, NS, LANES = _info.num_cores, _info.num_subcores, _info.num_lanes
NW = NC * NS
TOK_PER_W = N // NW
CHUNK = 1024
NCHUNK = TOK_PER_W // CHUNK


@functools.partial(
    pl.kernel,
    mesh=plsc.VectorSubcoreMesh(core_axis_name="c", subcore_axis_name="s"),
    out_type=jax.ShapeDtypeStruct((2 * N, BASE), jnp.float32),
    compiler_params=pltpu.CompilerParams(
        needs_layout_passes=False, use_tc_tiling_on_sc=False
    ),
    scratch_types=[
        pltpu.VMEM((CHUNK,), jnp.int32),
        pltpu.VMEM((CHUNK,), jnp.int32),
        pltpu.VMEM((CHUNK,), jnp.int32),
        pltpu.VMEM((2 * CHUNK,), jnp.int32),
        pltpu.VMEM((2 * CHUNK, BASE), jnp.float32),
        pltpu.SemaphoreType.DMA,
    ],
)
def _sc_gather(table_hbm, t1_hbm, t2_hbm, item_hbm, out_hbm,
               t1_v, t2_v, item_v, idx_v, rows_v, sem):
    wid = lax.axis_index("s") * NC + lax.axis_index("c")
    base = wid * TOK_PER_W
    lane = lax.iota(jnp.int32, LANES)

    def chunk_body(k, carry):
        off = base + k * CHUNK
        pltpu.sync_copy(t1_hbm.at[pl.ds(off, CHUNK)], t1_v)
        pltpu.sync_copy(t2_hbm.at[pl.ds(off, CHUNK)], t2_v)
        pltpu.sync_copy(item_hbm.at[pl.ds(off, CHUNK)], item_v)

        def vec_body(j, carry2):
            sl = pl.ds(j * LANES, LANES)
            it = item_v[sl] * 17
            m_idx = it + t1_v[sl]
            w_idx = it + (t2_v[sl] + T1)
            pos = lane * 2 + j * (2 * LANES)
            plsc.store_scatter(idx_v, [pos], m_idx)
            plsc.store_scatter(idx_v, [pos + 1], w_idx)
            return carry2

        lax.fori_loop(0, CHUNK // LANES, vec_body, 0)
        pltpu.async_copy(table_hbm.at[idx_v], rows_v, sem).wait()
        pltpu.sync_copy(rows_v, out_hbm.at[pl.ds(2 * off, 2 * CHUNK)])
        return carry

    lax.fori_loop(0, NCHUNK, chunk_body, 0)


_TW = 512
_TGRID = -(-C // _TW)


def _tc_transpose_body(m_ref, w_ref, o_ref):
    o_ref[:, 0:192] = m_ref[...].T
    o_ref[:, 192:272] = w_ref[...].T


_tc_transpose = pl.pallas_call(
    _tc_transpose_body,
    grid=(_TGRID,),
    in_specs=[
        pl.BlockSpec((192, _TW), lambda p: (0, p)),
        pl.BlockSpec((80, _TW), lambda p: (0, p)),
    ],
    out_specs=pl.BlockSpec((_TW, 272), lambda p: (p, 0)),
    out_shape=jax.ShapeDtypeStruct((C, 272), jnp.float32),
)


def kernel(log_seqs, time1_seqs, time2_seqs, month_pop_table, week_pop_table):
    table = _tc_transpose(month_pop_table, week_pop_table).reshape(17 * C, BASE)
    t1 = time1_seqs.reshape(-1).astype(jnp.int32)
    t2 = time2_seqs.reshape(-1).astype(jnp.int32)
    item = log_seqs.reshape(-1).astype(jnp.int32)
    rows = _sc_gather(table, t1, t2, item)
    return rows.reshape(B, L, 2 * BASE)


# speedup vs baseline: 4.4830x; 1.0315x over previous
import functools

import jax
import jax.numpy as jnp
from jax import lax
from jax.experimental import pallas as pl
from jax.experimental.pallas import tpu as pltpu
from jax.experimental.pallas import tpu_sc as plsc

B = 4096
L = 200
C = 100001
T1 = 12
BASE = 16
N = B * L

_info = plsc.get_sparse_core_info()
NC, NS, LANES = _info.num_cores, _info.num_subcores, _info.num_lanes
NW = NC * NS
TOK_PER_W = N // NW
CHUNK = 1024
NCHUNK = TOK_PER_W // CHUNK


@functools.partial(
    pl.kernel,
    mesh=plsc.VectorSubcoreMesh(core_axis_name="c", subcore_axis_name="s"),
    out_type=jax.ShapeDtypeStruct((2 * N, BASE), jnp.float32),
    compiler_params=pltpu.CompilerParams(
        needs_layout_passes=False, use_tc_tiling_on_sc=False
    ),
    scratch_types=[
        pltpu.VMEM((CHUNK,), jnp.int32),
        pltpu.VMEM((CHUNK,), jnp.int32),
        pltpu.VMEM((CHUNK,), jnp.int32),
        pltpu.VMEM((2 * CHUNK,), jnp.int32),
        pltpu.VMEM((2 * CHUNK, BASE), jnp.float32),
        pltpu.SemaphoreType.DMA,
    ],
)
def _sc_gather(table_hbm, t1_hbm, t2_hbm, item_hbm, out_hbm,
               t1_v, t2_v, item_v, idx_v, rows_v, sem):
    wid = lax.axis_index("s") * NC + lax.axis_index("c")
    base = wid * TOK_PER_W
    lane = lax.iota(jnp.int32, LANES)

    def chunk_body(k, carry):
        off = base + k * CHUNK
        pltpu.sync_copy(t1_hbm.at[pl.ds(off, CHUNK)], t1_v)
        pltpu.sync_copy(t2_hbm.at[pl.ds(off, CHUNK)], t2_v)
        pltpu.sync_copy(item_hbm.at[pl.ds(off, CHUNK)], item_v)

        def vec_body(j, carry2):
            sl = pl.ds(j * LANES, LANES)
            it = item_v[sl] * 24
            m_idx = it + t1_v[sl]
            w_idx = it + (t2_v[sl] + T1)
            pos = lane * 2 + j * (2 * LANES)
            plsc.store_scatter(idx_v, [pos], m_idx)
            plsc.store_scatter(idx_v, [pos + 1], w_idx)
            return carry2

        lax.fori_loop(0, CHUNK // LANES, vec_body, 0)
        pltpu.async_copy(table_hbm.at[idx_v], rows_v, sem).wait()
        pltpu.sync_copy(rows_v, out_hbm.at[pl.ds(2 * off, 2 * CHUNK)])
        return carry

    lax.fori_loop(0, NCHUNK, chunk_body, 0)


_TW = 512
_TGRID = -(-C // _TW)


def _tc_transpose_body(m_ref, w_ref, o_ref):
    o_ref[:, 0:128] = m_ref[0:128, :].T
    o_ref[:, 128:256] = jnp.concatenate(
        [m_ref[128:192, :], w_ref[0:64, :]], axis=0
    ).T
    o_ref[:, 256:272] = w_ref[64:80, :].T


_tc_transpose = pl.pallas_call(
    _tc_transpose_body,
    grid=(_TGRID,),
    in_specs=[
        pl.BlockSpec((192, _TW), lambda p: (0, p)),
        pl.BlockSpec((80, _TW), lambda p: (0, p)),
    ],
    out_specs=pl.BlockSpec((_TW, 384), lambda p: (p, 0)),
    out_shape=jax.ShapeDtypeStruct((C, 384), jnp.float32),
)


def kernel(log_seqs, time1_seqs, time2_seqs, month_pop_table, week_pop_table):
    table = _tc_transpose(month_pop_table, week_pop_table).reshape(24 * C, BASE)
    t1 = time1_seqs.reshape(-1).astype(jnp.int32)
    t2 = time2_seqs.reshape(-1).astype(jnp.int32)
    item = log_seqs.reshape(-1).astype(jnp.int32)
    rows = _sc_gather(table, t1, t2, item)
    return rows.reshape(B, L, 2 * BASE)


# speedup vs baseline: 5.2968x; 1.1815x over previous
import functools

import jax
import jax.numpy as jnp
from jax import lax
from jax.experimental import pallas as pl
from jax.experimental.pallas import tpu as pltpu
from jax.experimental.pallas import tpu_sc as plsc

B = 4096
L = 200
C = 100001
T1 = 12
BASE = 16
N = B * L

_TW = 512
_TGRID = -(-C // _TW)
CP = _TGRID * _TW

_info = plsc.get_sparse_core_info()
NC, NS, LANES = _info.num_cores, _info.num_subcores, _info.num_lanes
NW = NC * NS
TOK_PER_W = N // NW
CHUNK = 1024
NCHUNK = TOK_PER_W // CHUNK


@functools.partial(
    pl.kernel,
    mesh=plsc.VectorSubcoreMesh(core_axis_name="c", subcore_axis_name="s"),
    out_type=jax.ShapeDtypeStruct((2 * N, BASE), jnp.float32),
    compiler_params=pltpu.CompilerParams(
        needs_layout_passes=False, use_tc_tiling_on_sc=False
    ),
    scratch_types=[
        pltpu.VMEM((CHUNK,), jnp.int32),
        pltpu.VMEM((CHUNK,), jnp.int32),
        pltpu.VMEM((CHUNK,), jnp.int32),
        pltpu.VMEM((2 * CHUNK,), jnp.int32),
        pltpu.VMEM((2 * CHUNK, BASE), jnp.float32),
        pltpu.SemaphoreType.DMA,
    ],
)
def _sc_gather(table_hbm, t1_hbm, t2_hbm, item_hbm, out_hbm,
               t1_v, t2_v, item_v, idx_v, rows_v, sem):
    wid = lax.axis_index("s") * NC + lax.axis_index("c")
    base = wid * TOK_PER_W
    lane = lax.iota(jnp.int32, LANES)

    def chunk_body(k, carry):
        off = base + k * CHUNK
        pltpu.sync_copy(t1_hbm.at[pl.ds(off, CHUNK)], t1_v)
        pltpu.sync_copy(t2_hbm.at[pl.ds(off, CHUNK)], t2_v)
        pltpu.sync_copy(item_hbm.at[pl.ds(off, CHUNK)], item_v)

        def vec_body(j, carry2):
            sl = pl.ds(j * LANES, LANES)
            it8 = item_v[sl] * 8
            mk = t1_v[sl]
            wk = t2_v[sl] + T1
            m_idx = (mk >> 3) * (CP * 8) + it8 + (mk & 7)
            w_idx = (wk >> 3) * (CP * 8) + it8 + (wk & 7)
            pos = lane * 2 + j * (2 * LANES)
            plsc.store_scatter(idx_v, [pos], m_idx)
            plsc.store_scatter(idx_v, [pos + 1], w_idx)
            return carry2

        lax.fori_loop(0, CHUNK // LANES, vec_body, 0)
        pltpu.async_copy(table_hbm.at[idx_v], rows_v, sem).wait()
        pltpu.sync_copy(rows_v, out_hbm.at[pl.ds(2 * off, 2 * CHUNK)])
        return carry

    lax.fori_loop(0, NCHUNK, chunk_body, 0)


def _tc_transpose_body(m_ref, w_ref, o_ref):
    o_ref[0] = m_ref[0:128, :].T
    o_ref[1] = jnp.concatenate([m_ref[128:192, :], w_ref[0:64, :]], axis=0).T
    o_ref[2, :, 0:16] = w_ref[64:80, :].T


_tc_transpose = pl.pallas_call(
    _tc_transpose_body,
    grid=(_TGRID,),
    in_specs=[
        pl.BlockSpec((192, _TW), lambda p: (0, p)),
        pl.BlockSpec((80, _TW), lambda p: (0, p)),
    ],
    out_specs=pl.BlockSpec((3, _TW, 128), lambda p: (0, p, 0)),
    out_shape=jax.ShapeDtypeStruct((3, CP, 128), jnp.float32),
)


def kernel(log_seqs, time1_seqs, time2_seqs, month_pop_table, week_pop_table):
    table = _tc_transpose(month_pop_table, week_pop_table).reshape(3 * CP * 8, BASE)
    t1 = time1_seqs.reshape(-1).astype(jnp.int32)
    t2 = time2_seqs.reshape(-1).astype(jnp.int32)
    item = log_seqs.reshape(-1).astype(jnp.int32)
    rows = _sc_gather(table, t1, t2, item)
    return rows.reshape(B, L, 2 * BASE)


# speedup vs baseline: 5.7979x; 1.0946x over previous
import functools

import jax
import jax.numpy as jnp
from jax import lax
from jax.experimental import pallas as pl
from jax.experimental.pallas import tpu as pltpu
from jax.experimental.pallas import tpu_sc as plsc

B = 4096
L = 200
C = 100001
T1 = 12
BASE = 16
N = B * L

_TW = 512
_TGRID = -(-C // _TW)
CP = _TGRID * _TW

_info = plsc.get_sparse_core_info()
NC, NS, LANES = _info.num_cores, _info.num_subcores, _info.num_lanes
NW = NC * NS
TOK_PER_W = N // NW
CHUNK = 1024
NCHUNK = TOK_PER_W // CHUNK


@functools.partial(
    pl.kernel,
    mesh=plsc.VectorSubcoreMesh(core_axis_name="c", subcore_axis_name="s"),
    out_type=jax.ShapeDtypeStruct((2 * N, BASE), jnp.float32),
    compiler_params=pltpu.CompilerParams(
        needs_layout_passes=False, use_tc_tiling_on_sc=False
    ),
    scratch_types=[
        pltpu.VMEM((CHUNK,), jnp.int32),
        pltpu.VMEM((CHUNK,), jnp.int32),
        pltpu.VMEM((CHUNK,), jnp.int32),
        pltpu.VMEM((2 * CHUNK,), jnp.int32),
        pltpu.VMEM((2 * CHUNK, BASE), jnp.float32),
        pltpu.SemaphoreType.DMA,
    ],
)
def _sc_gather(table_hbm, t1_hbm, t2_hbm, item_hbm, out_hbm,
               t1_v, t2_v, item_v, idx_v, rows_v, sem):
    wid = lax.axis_index("s") * NC + lax.axis_index("c")
    base = wid * TOK_PER_W
    lane = lax.iota(jnp.int32, LANES)

    def chunk_body(k, carry):
        off = base + k * CHUNK
        pltpu.sync_copy(t1_hbm.at[pl.ds(off, CHUNK)], t1_v)
        pltpu.sync_copy(t2_hbm.at[pl.ds(off, CHUNK)], t2_v)
        pltpu.sync_copy(item_hbm.at[pl.ds(off, CHUNK)], item_v)

        def vec_body(j, carry2):
            sl = pl.ds(j * LANES, LANES)
            it8 = item_v[sl] * 8
            mk = t1_v[sl]
            wk = t2_v[sl] + T1
            m_idx = (mk >> 3) * (CP * 8) + it8 + (mk & 7)
            w_idx = (wk >> 3) * (CP * 8) + it8 + (wk & 7)
            pos = lane * 2 + j * (2 * LANES)
            plsc.store_scatter(idx_v, [pos], m_idx)
            plsc.store_scatter(idx_v, [pos + 1], w_idx)
            return carry2

        lax.fori_loop(0, CHUNK // LANES, vec_body, 0)
        pltpu.async_copy(table_hbm.at[idx_v], rows_v, sem).wait()
        pltpu.sync_copy(rows_v, out_hbm.at[pl.ds(2 * off, 2 * CHUNK)])
        return carry

    lax.fori_loop(0, NCHUNK, chunk_body, 0)


def _tc_transpose_body(m_ref, w_ref, o_ref):
    o_ref[0] = m_ref[0:128, :].T
    o_ref[1] = jnp.concatenate([m_ref[128:192, :], w_ref[0:64, :]], axis=0).T
    o_ref[2, :, 0:16] = w_ref[64:80, :].T


_tc_transpose = pl.pallas_call(
    _tc_transpose_body,
    grid=(_TGRID,),
    in_specs=[
        pl.BlockSpec((192, _TW), lambda p: (0, p)),
        pl.BlockSpec((80, _TW), lambda p: (0, p)),
    ],
    out_specs=pl.BlockSpec((3, _TW, 128), lambda p: (0, p, 0)),
    out_shape=jax.ShapeDtypeStruct((3, CP, 128), jnp.float32),
)


def kernel(log_seqs, time1_seqs, time2_seqs, month_pop_table, week_pop_table):
    table = _tc_transpose(month_pop_table, week_pop_table).reshape(3 * CP * 8, BASE)
    t1 = time1_seqs.T.reshape(-1).astype(jnp.int32)
    t2 = time2_seqs.T.reshape(-1).astype(jnp.int32)
    item = log_seqs.T.reshape(-1).astype(jnp.int32)
    rows = _sc_gather(table, t1, t2, item)
    return rows.reshape(L, B, 2 * BASE).transpose(1, 0, 2)


# speedup vs baseline: 6.6410x; 1.1454x over previous
import functools

import jax
import jax.numpy as jnp
from jax import lax
from jax.experimental import pallas as pl
from jax.experimental.pallas import tpu as pltpu
from jax.experimental.pallas import tpu_sc as plsc

B = 4096
L = 200
C = 100001
T1 = 12
BASE = 16
N = B * L

_TW = 512
_TGRID = -(-C // _TW)
CP = _TGRID * _TW

_info = plsc.get_sparse_core_info()
NC, NS, LANES = _info.num_cores, _info.num_subcores, _info.num_lanes
NW = NC * NS
TOK_PER_W = N // NW
CHUNK = 1024
NCHUNK = TOK_PER_W // CHUNK


@functools.partial(
    pl.kernel,
    mesh=plsc.VectorSubcoreMesh(core_axis_name="c", subcore_axis_name="s"),
    out_type=jax.ShapeDtypeStruct((2 * N, BASE), jnp.float32),
    compiler_params=pltpu.CompilerParams(
        needs_layout_passes=False, use_tc_tiling_on_sc=False
    ),
    scratch_types=[
        pltpu.VMEM((CHUNK,), jnp.int32),
        pltpu.VMEM((CHUNK,), jnp.int32),
        pltpu.VMEM((CHUNK,), jnp.int32),
        pltpu.VMEM((2 * CHUNK,), jnp.int32),
        pltpu.VMEM((2 * CHUNK, BASE), jnp.float32),
        pltpu.SemaphoreType.DMA,
    ],
)
def _sc_gather(table_hbm, t1_hbm, t2_hbm, item_hbm, out_hbm,
               t1_v, t2_v, item_v, idx_v, rows_v, sem):
    wid = lax.axis_index("s") * NC + lax.axis_index("c")
    lane = lax.iota(jnp.int32, LANES)
    p0 = (lane & 3) * 256 + (lane >> 2)

    def chunk_body(k, carry):
        cid = wid * NCHUNK + k
        lq = cid >> 2
        q = cid & 3
        pbase = lq * B + q * 256
        for g in range(4):
            run = pl.ds(pbase + g * 1024, 256)
            dst = pl.ds(g * 256, 256)
            pltpu.sync_copy(t1_hbm.at[run], t1_v.at[dst])
            pltpu.sync_copy(t2_hbm.at[run], t2_v.at[dst])
            pltpu.sync_copy(item_hbm.at[run], item_v.at[dst])

        def vec_body(j, carry2):
            pos_in = p0 + 4 * j
            it8 = plsc.load_gather(item_v, [pos_in]) * 8
            mk = plsc.load_gather(t1_v, [pos_in])
            wk = plsc.load_gather(t2_v, [pos_in]) + T1
            m_idx = (mk >> 3) * (CP * 8) + it8 + (mk & 7)
            w_idx = (wk >> 3) * (CP * 8) + it8 + (wk & 7)
            pos = lane * 2 + j * (2 * LANES)
            plsc.store_scatter(idx_v, [pos], m_idx)
            plsc.store_scatter(idx_v, [pos + 1], w_idx)
            return carry2

        lax.fori_loop(0, CHUNK // LANES, vec_body, 0)
        pltpu.async_copy(table_hbm.at[idx_v], rows_v, sem).wait()
        pltpu.sync_copy(rows_v, out_hbm.at[pl.ds(2 * cid * CHUNK, 2 * CHUNK)])
        return carry

    lax.fori_loop(0, NCHUNK, chunk_body, 0)


def _tc_transpose_body(m_ref, w_ref, o_ref):
    o_ref[0] = m_ref[0:128, :].T
    o_ref[1] = jnp.concatenate([m_ref[128:192, :], w_ref[0:64, :]], axis=0).T
    o_ref[2, :, 0:16] = w_ref[64:80, :].T


_tc_transpose = pl.pallas_call(
    _tc_transpose_body,
    grid=(_TGRID,),
    in_specs=[
        pl.BlockSpec((192, _TW), lambda p: (0, p)),
        pl.BlockSpec((80, _TW), lambda p: (0, p)),
    ],
    out_specs=pl.BlockSpec((3, _TW, 128), lambda p: (0, p, 0)),
    out_shape=jax.ShapeDtypeStruct((3, CP, 128), jnp.float32),
)


def _tc_relayout_body(x_ref, o_ref):
    z = x_ref[0].T
    for g in range(4):
        o_ref[0, :, pl.ds(g * 1024, 1024)] = z[32 * g:32 * (g + 1), :]


_tc_relayout = pl.pallas_call(
    _tc_relayout_body,
    grid=(L,),
    in_specs=[pl.BlockSpec((1, 1024, 128), lambda l: (l, 0, 0))],
    out_specs=pl.BlockSpec((1, 2 * BASE, B), lambda l: (l, 0, 0)),
    out_shape=jax.ShapeDtypeStruct((L, 2 * BASE, B), jnp.float32),
)


def kernel(log_seqs, time1_seqs, time2_seqs, month_pop_table, week_pop_table):
    table = _tc_transpose(month_pop_table, week_pop_table).reshape(3 * CP * 8, BASE)
    t1 = time1_seqs.T.reshape(-1).astype(jnp.int32)
    t2 = time2_seqs.T.reshape(-1).astype(jnp.int32)
    item = log_seqs.T.reshape(-1).astype(jnp.int32)
    rows = _sc_gather(table, t1, t2, item)
    planes = _tc_relayout(rows.reshape(L, 1024, 128))
    return planes.transpose(2, 0, 1)


# speedup vs baseline: 8.3631x; 1.2593x over previous
import functools

import jax
import jax.numpy as jnp
from jax import lax
from jax.experimental import pallas as pl
from jax.experimental.pallas import tpu as pltpu
from jax.experimental.pallas import tpu_sc as plsc

B = 4096
L = 200
C = 100001
T1 = 12
BASE = 16
N = B * L

_TW = 512
_TGRID = -(-C // _TW)
CP = _TGRID * _TW

_info = plsc.get_sparse_core_info()
NC, NS, LANES = _info.num_cores, _info.num_subcores, _info.num_lanes
NW = NC * NS
TOK_PER_W = N // NW
CHUNK = 1024
NCHUNK = TOK_PER_W // CHUNK


@functools.partial(
    pl.kernel,
    mesh=plsc.VectorSubcoreMesh(core_axis_name="c", subcore_axis_name="s"),
    out_type=jax.ShapeDtypeStruct((2 * N, BASE), jnp.float32),
    compiler_params=pltpu.CompilerParams(
        needs_layout_passes=False, use_tc_tiling_on_sc=False
    ),
    scratch_types=[
        pltpu.VMEM((CHUNK,), jnp.int32),
        pltpu.VMEM((CHUNK,), jnp.int32),
        pltpu.VMEM((CHUNK,), jnp.int32),
        pltpu.VMEM((2 * CHUNK,), jnp.int32),
        pltpu.VMEM((2 * CHUNK, BASE), jnp.float32),
        pltpu.SemaphoreType.DMA,
    ],
)
def _sc_gather(table_hbm, t1_hbm, t2_hbm, item_hbm, out_hbm,
               t1_v, t2_v, item_v, idx_v, rows_v, sem):
    wid = lax.axis_index("s") * NC + lax.axis_index("c")
    lane = lax.iota(jnp.int32, LANES)
    p0 = (lane & 3) * 256 + (lane >> 2)

    def chunk_body(k, carry):
        cid = wid * NCHUNK + k
        lq = cid >> 2
        q = cid & 3
        pbase = lq * B + q * 256
        handles = []
        for g in range(4):
            run = pl.ds(pbase + g * 1024, 256)
            dst = pl.ds(g * 256, 256)
            handles.append(pltpu.async_copy(t1_hbm.at[run], t1_v.at[dst], sem))
            handles.append(pltpu.async_copy(t2_hbm.at[run], t2_v.at[dst], sem))
            handles.append(pltpu.async_copy(item_hbm.at[run], item_v.at[dst], sem))
        for h in handles:
            h.wait()

        def vec_body(j, carry2):
            pos_in = p0 + 4 * j
            it8 = plsc.load_gather(item_v, [pos_in]) * 8
            mk = plsc.load_gather(t1_v, [pos_in])
            wk = plsc.load_gather(t2_v, [pos_in]) + T1
            m_idx = (mk >> 3) * (CP * 8) + it8 + (mk & 7)
            w_idx = (wk >> 3) * (CP * 8) + it8 + (wk & 7)
            pos = lane * 2 + j * (2 * LANES)
            plsc.store_scatter(idx_v, [pos], m_idx)
            plsc.store_scatter(idx_v, [pos + 1], w_idx)
            return carry2

        lax.fori_loop(0, CHUNK // LANES, vec_body, 0)
        pltpu.async_copy(table_hbm.at[idx_v], rows_v, sem).wait()
        pltpu.sync_copy(rows_v, out_hbm.at[pl.ds(2 * cid * CHUNK, 2 * CHUNK)])
        return carry

    lax.fori_loop(0, NCHUNK, chunk_body, 0)


def _tc_transpose_body(m_ref, w_ref, o_ref):
    o_ref[0] = m_ref[0:128, :].T
    o_ref[1] = jnp.concatenate([m_ref[128:192, :], w_ref[0:64, :]], axis=0).T
    o_ref[2, :, 0:16] = w_ref[64:80, :].T


_tc_transpose = pl.pallas_call(
    _tc_transpose_body,
    grid=(_TGRID,),
    in_specs=[
        pl.BlockSpec((192, _TW), lambda p: (0, p)),
        pl.BlockSpec((80, _TW), lambda p: (0, p)),
    ],
    out_specs=pl.BlockSpec((3, _TW, 128), lambda p: (0, p, 0)),
    out_shape=jax.ShapeDtypeStruct((3, CP, 128), jnp.float32),
)


def _tc_relayout_body(x_ref, o_ref):
    z = x_ref[0].T
    for g in range(4):
        o_ref[0, :, pl.ds(g * 1024, 1024)] = z[32 * g:32 * (g + 1), :]


_tc_relayout = pl.pallas_call(
    _tc_relayout_body,
    grid=(L,),
    in_specs=[pl.BlockSpec((1, 1024, 128), lambda l: (l, 0, 0))],
    out_specs=pl.BlockSpec((1, 2 * BASE, B), lambda l: (l, 0, 0)),
    out_shape=jax.ShapeDtypeStruct((L, 2 * BASE, B), jnp.float32),
)


def kernel(log_seqs, time1_seqs, time2_seqs, month_pop_table, week_pop_table):
    table = _tc_transpose(month_pop_table, week_pop_table).reshape(3 * CP * 8, BASE)
    t1 = time1_seqs.T.reshape(-1).astype(jnp.int32)
    t2 = time2_seqs.T.reshape(-1).astype(jnp.int32)
    item = log_seqs.T.reshape(-1).astype(jnp.int32)
    rows = _sc_gather(table, t1, t2, item)
    planes = _tc_relayout(rows.reshape(L, 1024, 128))
    return planes.transpose(2, 0, 1)


# speedup vs baseline: 10.2684x; 1.2278x over previous
import functools

import jax
import jax.numpy as jnp
from jax import lax
from jax.experimental import pallas as pl
from jax.experimental.pallas import tpu as pltpu
from jax.experimental.pallas import tpu_sc as plsc

B = 4096
L = 200
C = 100001
T1 = 12
BASE = 16
N = B * L

_TW = 1024
_TGRID = -(-C // _TW)
CP = _TGRID * _TW

_info = plsc.get_sparse_core_info()
NC, NS, LANES = _info.num_cores, _info.num_subcores, _info.num_lanes
NW = NC * NS
TOK_PER_W = N // NW
CHUNK = 1024
NCHUNK = TOK_PER_W // CHUNK


@functools.partial(
    pl.kernel,
    mesh=plsc.VectorSubcoreMesh(core_axis_name="c", subcore_axis_name="s"),
    out_type=jax.ShapeDtypeStruct((2 * N, BASE), jnp.float32),
    compiler_params=pltpu.CompilerParams(
        needs_layout_passes=False, use_tc_tiling_on_sc=False
    ),
    scratch_types=[
        pltpu.VMEM((CHUNK,), jnp.int32),
        pltpu.VMEM((CHUNK,), jnp.int32),
        pltpu.VMEM((CHUNK,), jnp.int32),
        pltpu.VMEM((2 * CHUNK,), jnp.int32),
        pltpu.VMEM((2 * CHUNK, BASE), jnp.float32),
        pltpu.SemaphoreType.DMA,
    ],
)
def _sc_gather(table_hbm, t1_hbm, t2_hbm, item_hbm, out_hbm,
               t1_v, t2_v, item_v, idx_v, rows_v, sem):
    wid = lax.axis_index("s") * NC + lax.axis_index("c")
    lane = lax.iota(jnp.int32, LANES)
    p0 = (lane & 3) * 256 + (lane >> 2)

    def chunk_body(k, carry):
        cid = wid * NCHUNK + k
        lq = cid >> 2
        q = cid & 3
        pbase = lq * B + q * 256
        handles = []
        for g in range(4):
            run = pl.ds(pbase + g * 1024, 256)
            dst = pl.ds(g * 256, 256)
            handles.append(pltpu.async_copy(t1_hbm.at[run], t1_v.at[dst], sem))
            handles.append(pltpu.async_copy(t2_hbm.at[run], t2_v.at[dst], sem))
            handles.append(pltpu.async_copy(item_hbm.at[run], item_v.at[dst], sem))
        for h in handles:
            h.wait()

        def vec_body(j, carry2):
            pos_in = p0 + 4 * j
            it8 = plsc.load_gather(item_v, [pos_in]) * 8
            mk = plsc.load_gather(t1_v, [pos_in])
            wk = plsc.load_gather(t2_v, [pos_in]) + T1
            m_idx = (mk >> 3) * (CP * 8) + it8 + (mk & 7)
            w_idx = (wk >> 3) * (CP * 8) + it8 + (wk & 7)
            pos = lane * 2 + j * (2 * LANES)
            plsc.store_scatter(idx_v, [pos], m_idx)
            plsc.store_scatter(idx_v, [pos + 1], w_idx)
            return carry2

        lax.fori_loop(0, CHUNK // LANES, vec_body, 0)
        pltpu.async_copy(table_hbm.at[idx_v], rows_v, sem).wait()
        pltpu.sync_copy(rows_v, out_hbm.at[pl.ds(2 * cid * CHUNK, 2 * CHUNK)])
        return carry

    lax.fori_loop(0, NCHUNK, chunk_body, 0)


def _tc_transpose_body(m_ref, w_ref, o_ref):
    o_ref[0] = m_ref[0:128, :].T
    o_ref[1] = jnp.concatenate([m_ref[128:192, :], w_ref[0:64, :]], axis=0).T
    o_ref[2, :, 0:16] = w_ref[64:80, :].T


_tc_transpose = pl.pallas_call(
    _tc_transpose_body,
    grid=(_TGRID,),
    in_specs=[
        pl.BlockSpec((192, _TW), lambda p: (0, p)),
        pl.BlockSpec((80, _TW), lambda p: (0, p)),
    ],
    out_specs=pl.BlockSpec((3, _TW, 128), lambda p: (0, p, 0)),
    out_shape=jax.ShapeDtypeStruct((3, CP, 128), jnp.float32),
)


def _tc_relayout_body(x_ref, o_ref):
    for i in range(2):
        z = x_ref[i].T
        for g in range(4):
            o_ref[i, :, pl.ds(g * 1024, 1024)] = z[32 * g:32 * (g + 1), :]


_tc_relayout = pl.pallas_call(
    _tc_relayout_body,
    grid=(L // 2,),
    in_specs=[pl.BlockSpec((2, 1024, 128), lambda l: (l, 0, 0))],
    out_specs=pl.BlockSpec((2, 2 * BASE, B), lambda l: (l, 0, 0)),
    out_shape=jax.ShapeDtypeStruct((L, 2 * BASE, B), jnp.float32),
)


def kernel(log_seqs, time1_seqs, time2_seqs, month_pop_table, week_pop_table):
    table = _tc_transpose(month_pop_table, week_pop_table).reshape(3 * CP * 8, BASE)
    t1 = time1_seqs.T.reshape(-1).astype(jnp.int32)
    t2 = time2_seqs.T.reshape(-1).astype(jnp.int32)
    item = log_seqs.T.reshape(-1).astype(jnp.int32)
    rows = _sc_gather(table, t1, t2, item)
    planes = _tc_relayout(rows.reshape(L, 1024, 128))
    return planes.transpose(2, 0, 1)


# speedup vs baseline: 11.9895x; 1.1676x over previous
import functools

import jax
import jax.numpy as jnp
from jax import lax
from jax.experimental import pallas as pl
from jax.experimental.pallas import tpu as pltpu
from jax.experimental.pallas import tpu_sc as plsc

B = 4096
L = 200
C = 100001
T1 = 12
BASE = 16
N = B * L

_TW = 2048
_TGRID = -(-C // _TW)
CP = _TGRID * _TW

_info = plsc.get_sparse_core_info()
NC, NS, LANES = _info.num_cores, _info.num_subcores, _info.num_lanes
NW = NC * NS
TOK_PER_W = N // NW
CHUNK = 1024
NCHUNK = TOK_PER_W // CHUNK


@functools.partial(
    pl.kernel,
    mesh=plsc.VectorSubcoreMesh(core_axis_name="c", subcore_axis_name="s"),
    out_type=jax.ShapeDtypeStruct((2 * N, BASE), jnp.float32),
    compiler_params=pltpu.CompilerParams(
        needs_layout_passes=False, use_tc_tiling_on_sc=False
    ),
    scratch_types=[
        pltpu.VMEM((CHUNK,), jnp.int32),
        pltpu.VMEM((CHUNK,), jnp.int32),
        pltpu.VMEM((CHUNK,), jnp.int32),
        pltpu.VMEM((2 * CHUNK,), jnp.int32),
        pltpu.VMEM((2 * CHUNK, BASE), jnp.float32),
        pltpu.SemaphoreType.DMA,
    ],
)
def _sc_gather(table_hbm, t1_hbm, t2_hbm, item_hbm, out_hbm,
               t1_v, t2_v, item_v, idx_v, rows_v, sem):
    wid = lax.axis_index("s") * NC + lax.axis_index("c")
    lane = lax.iota(jnp.int32, LANES)
    p0 = (lane & 3) * 256 + (lane >> 2)

    def chunk_body(k, carry):
        cid = wid * NCHUNK + k
        lq = cid >> 2
        q = cid & 3
        pbase = lq * B + q * 256
        handles = []
        for g in range(4):
            run = pl.ds(pbase + g * 1024, 256)
            dst = pl.ds(g * 256, 256)
            handles.append(pltpu.async_copy(t1_hbm.at[run], t1_v.at[dst], sem))
            handles.append(pltpu.async_copy(t2_hbm.at[run], t2_v.at[dst], sem))
            handles.append(pltpu.async_copy(item_hbm.at[run], item_v.at[dst], sem))
        for h in handles:
            h.wait()

        def vec_body(j, carry2):
            pos_in = p0 + 4 * j
            it8 = plsc.load_gather(item_v, [pos_in]) * 8
            mk = plsc.load_gather(t1_v, [pos_in])
            wk = plsc.load_gather(t2_v, [pos_in]) + T1
            m_idx = (mk >> 3) * (CP * 8) + it8 + (mk & 7)
            w_idx = (wk >> 3) * (CP * 8) + it8 + (wk & 7)
            pos = lane * 2 + j * (2 * LANES)
            plsc.store_scatter(idx_v, [pos], m_idx)
            plsc.store_scatter(idx_v, [pos + 1], w_idx)
            return carry2

        lax.fori_loop(0, CHUNK // LANES, vec_body, 0)
        pltpu.async_copy(table_hbm.at[idx_v], rows_v, sem).wait()
        pltpu.sync_copy(rows_v, out_hbm.at[pl.ds(2 * cid * CHUNK, 2 * CHUNK)])
        return carry

    lax.fori_loop(0, NCHUNK, chunk_body, 0)


def _tc_transpose_body(m_ref, w_ref, o_ref):
    o_ref[0] = m_ref[0:128, :].T
    o_ref[1] = jnp.concatenate([m_ref[128:192, :], w_ref[0:64, :]], axis=0).T
    o_ref[2, :, 0:16] = w_ref[64:80, :].T


_tc_transpose = pl.pallas_call(
    _tc_transpose_body,
    grid=(_TGRID,),
    in_specs=[
        pl.BlockSpec((192, _TW), lambda p: (0, p)),
        pl.BlockSpec((80, _TW), lambda p: (0, p)),
    ],
    out_specs=pl.BlockSpec((3, _TW, 128), lambda p: (0, p, 0)),
    out_shape=jax.ShapeDtypeStruct((3, CP, 128), jnp.float32),
)


def _tc_relayout_body(x_ref, o_ref):
    for i in range(4):
        z = x_ref[i].T
        for g in range(4):
            o_ref[i, :, pl.ds(g * 1024, 1024)] = z[32 * g:32 * (g + 1), :]


_tc_relayout = pl.pallas_call(
    _tc_relayout_body,
    grid=(L // 4,),
    in_specs=[pl.BlockSpec((4, 1024, 128), lambda l: (l, 0, 0))],
    out_specs=pl.BlockSpec((4, 2 * BASE, B), lambda l: (l, 0, 0)),
    out_shape=jax.ShapeDtypeStruct((L, 2 * BASE, B), jnp.float32),
)


def kernel(log_seqs, time1_seqs, time2_seqs, month_pop_table, week_pop_table):
    table = _tc_transpose(month_pop_table, week_pop_table).reshape(3 * CP * 8, BASE)
    t1 = time1_seqs.T.reshape(-1).astype(jnp.int32)
    t2 = time2_seqs.T.reshape(-1).astype(jnp.int32)
    item = log_seqs.T.reshape(-1).astype(jnp.int32)
    rows = _sc_gather(table, t1, t2, item)
    planes = _tc_relayout(rows.reshape(L, 1024, 128))
    return planes.transpose(2, 0, 1)


# speedup vs baseline: 14.4263x; 1.2032x over previous
import functools

import jax
import jax.numpy as jnp
from jax import lax
from jax.experimental import pallas as pl
from jax.experimental.pallas import tpu as pltpu
from jax.experimental.pallas import tpu_sc as plsc

B = 4096
L = 200
C = 100001
T1 = 12
BASE = 16
N = B * L

_TW = 2048
_TGRID = -(-C // _TW)
CP = _TGRID * _TW

_info = plsc.get_sparse_core_info()
NC, NS, LANES = _info.num_cores, _info.num_subcores, _info.num_lanes
NW = NC * NS
TOK_PER_W = N // NW
CHUNK = 1024
NCHUNK = TOK_PER_W // CHUNK


@functools.partial(
    pl.kernel,
    mesh=plsc.VectorSubcoreMesh(core_axis_name="c", subcore_axis_name="s"),
    out_type=jax.ShapeDtypeStruct((2 * N, BASE), jnp.float32),
    compiler_params=pltpu.CompilerParams(
        needs_layout_passes=False, use_tc_tiling_on_sc=False
    ),
    scratch_types=[
        pltpu.VMEM((2, 3, CHUNK), jnp.int32),
        pltpu.VMEM((2, 2 * CHUNK), jnp.int32),
        pltpu.VMEM((2, 2 * CHUNK, BASE), jnp.float32),
        pltpu.SemaphoreType.DMA,
        pltpu.SemaphoreType.DMA,
        pltpu.SemaphoreType.DMA,
        pltpu.SemaphoreType.DMA,
    ],
)
def _sc_gather(table_hbm, t1_hbm, t2_hbm, item_hbm, out_hbm,
               tin, idx2, rows2, sem_in0, sem_in1, sem_g, sem_out):
    wid = lax.axis_index("s") * NC + lax.axis_index("c")
    lane = lax.iota(jnp.int32, LANES)
    p0 = (lane & 3) * 256 + (lane >> 2)

    def in_copies(k, buf):
        cid = wid * NCHUNK + k
        pbase = (cid >> 2) * B + (cid & 3) * 256
        sem_in = sem_in1 if buf else sem_in0
        out = []
        for g in range(4):
            run = pl.ds(pbase + g * 1024, 256)
            for ai, src in enumerate((t1_hbm, t2_hbm, item_hbm)):
                out.append(pltpu.make_async_copy(
                    src.at[run], tin.at[buf, ai, pl.ds(g * 256, 256)], sem_in))
        return out

    def out_copy(k, buf):
        cid = wid * NCHUNK + k
        return pltpu.make_async_copy(
            rows2.at[buf], out_hbm.at[pl.ds(2 * cid * CHUNK, 2 * CHUNK)],
            sem_out)

    def gather_copy(buf):
        return pltpu.make_async_copy(
            table_hbm.at[idx2.at[buf]], rows2.at[buf], sem_g)

    def build_idx(buf):
        def vec_body(j, carry2):
            pos_in = p0 + 4 * j
            it8 = plsc.load_gather(tin.at[buf, 2], [pos_in]) * 8
            mk = plsc.load_gather(tin.at[buf, 0], [pos_in])
            wk = plsc.load_gather(tin.at[buf, 1], [pos_in]) + T1
            m_idx = (mk >> 3) * (CP * 8) + it8 + (mk & 7)
            w_idx = (wk >> 3) * (CP * 8) + it8 + (wk & 7)
            pos = lane * 2 + j * (2 * LANES)
            plsc.store_scatter(idx2.at[buf], [pos], m_idx)
            plsc.store_scatter(idx2.at[buf], [pos + 1], w_idx)
            return carry2

        lax.fori_loop(0, CHUNK // LANES, vec_body, 0)

    for h in in_copies(0, 0):
        h.start()

    def loop_body(i, carry):
        for par in range(2):
            k = 2 * i + par
            buf = par
            nbuf = 1 - par

            @pl.when(k < NCHUNK)
            def _():
                @pl.when(k + 1 < NCHUNK)
                def _():
                    for h in in_copies(k + 1, nbuf):
                        h.start()

                for h in in_copies(k, buf):
                    h.wait()

                @pl.when(k >= 2)
                def _():
                    out_copy(k - 2, buf).wait()

                build_idx(buf)
                gather_copy(buf).start()

                @pl.when(k >= 1)
                def _():
                    gather_copy(nbuf).wait()
                    out_copy(k - 1, nbuf).start()

        return carry

    lax.fori_loop(0, (NCHUNK + 1) // 2, loop_body, 0)
    last = NCHUNK - 1
    gather_copy(last & 1).wait()
    out_copy(last, last & 1).start()
    out_copy(last - 1, 1 - (last & 1)).wait()
    out_copy(last, last & 1).wait()


def _tc_transpose_body(m_ref, w_ref, o_ref):
    o_ref[0] = m_ref[0:128, :].T
    o_ref[1] = jnp.concatenate([m_ref[128:192, :], w_ref[0:64, :]], axis=0).T
    o_ref[2, :, 0:16] = w_ref[64:80, :].T


_tc_transpose = pl.pallas_call(
    _tc_transpose_body,
    grid=(_TGRID,),
    in_specs=[
        pl.BlockSpec((192, _TW), lambda p: (0, p)),
        pl.BlockSpec((80, _TW), lambda p: (0, p)),
    ],
    out_specs=pl.BlockSpec((3, _TW, 128), lambda p: (0, p, 0)),
    out_shape=jax.ShapeDtypeStruct((3, CP, 128), jnp.float32),
)


def _tc_relayout_body(x_ref, o_ref):
    for i in range(4):
        z = x_ref[i].T
        for g in range(4):
            o_ref[i, :, pl.ds(g * 1024, 1024)] = z[32 * g:32 * (g + 1), :]


_tc_relayout = pl.pallas_call(
    _tc_relayout_body,
    grid=(L // 4,),
    in_specs=[pl.BlockSpec((4, 1024, 128), lambda l: (l, 0, 0))],
    out_specs=pl.BlockSpec((4, 2 * BASE, B), lambda l: (l, 0, 0)),
    out_shape=jax.ShapeDtypeStruct((L, 2 * BASE, B), jnp.float32),
)


def kernel(log_seqs, time1_seqs, time2_seqs, month_pop_table, week_pop_table):
    table = _tc_transpose(month_pop_table, week_pop_table).reshape(3 * CP * 8, BASE)
    t1 = time1_seqs.T.reshape(-1).astype(jnp.int32)
    t2 = time2_seqs.T.reshape(-1).astype(jnp.int32)
    item = log_seqs.T.reshape(-1).astype(jnp.int32)
    rows = _sc_gather(table, t1, t2, item)
    planes = _tc_relayout(rows.reshape(L, 1024, 128))
    return planes.transpose(2, 0, 1)


# speedup vs baseline: 15.0491x; 1.0432x over previous
import functools

import jax
import jax.numpy as jnp
from jax import lax
from jax.experimental import pallas as pl
from jax.experimental.pallas import tpu as pltpu
from jax.experimental.pallas import tpu_sc as plsc

B = 4096
L = 200
C = 100001
T1 = 12
BASE = 16
N = B * L

_TW = 2048
_TGRID = -(-C // _TW)
CP = _TGRID * _TW

_info = plsc.get_sparse_core_info()
NC, NS, LANES = _info.num_cores, _info.num_subcores, _info.num_lanes
NW = NC * NS
TOK_PER_W = N // NW
CHUNK = 1024
NCHUNK = TOK_PER_W // CHUNK


@functools.partial(
    pl.kernel,
    mesh=plsc.VectorSubcoreMesh(core_axis_name="c", subcore_axis_name="s"),
    out_type=jax.ShapeDtypeStruct((2 * N, BASE), jnp.float32),
    compiler_params=pltpu.CompilerParams(
        needs_layout_passes=False, use_tc_tiling_on_sc=False
    ),
    scratch_types=[
        pltpu.VMEM((2, 3, CHUNK), jnp.int32),
        pltpu.VMEM((2, 2 * CHUNK), jnp.int32),
        pltpu.VMEM((2, 2 * CHUNK, BASE), jnp.float32),
        pltpu.SemaphoreType.DMA,
        pltpu.SemaphoreType.DMA,
        pltpu.SemaphoreType.DMA,
        pltpu.SemaphoreType.DMA,
    ],
)
def _sc_gather(table_hbm, t1_hbm, t2_hbm, item_hbm, out_hbm,
               tin, idx2, rows2, sem_in0, sem_in1, sem_g, sem_out):
    wid = lax.axis_index("s") * NC + lax.axis_index("c")
    lane = lax.iota(jnp.int32, LANES)
    p0 = (lane & 3) * 256 + (lane >> 2)

    def in_copies(k, buf):
        cid = wid * NCHUNK + k
        pbase = (cid >> 2) * B + (cid & 3) * 256
        sem_in = sem_in1 if buf else sem_in0
        out = []
        for g in range(4):
            run = pl.ds(pbase + g * 1024, 256)
            for ai, src in enumerate((t1_hbm, t2_hbm, item_hbm)):
                out.append(pltpu.make_async_copy(
                    src.at[run], tin.at[buf, ai, pl.ds(g * 256, 256)], sem_in))
        return out

    def out_copy(k, buf):
        cid = wid * NCHUNK + k
        return pltpu.make_async_copy(
            rows2.at[buf], out_hbm.at[pl.ds(2 * cid * CHUNK, 2 * CHUNK)],
            sem_out)

    def gather_copy(buf):
        return pltpu.make_async_copy(
            table_hbm.at[idx2.at[buf]], rows2.at[buf], sem_g)

    def build_idx(buf):
        def vec_body(j, carry2):
            pos_in = p0 + 4 * j
            it8 = plsc.load_gather(tin.at[buf, 2], [pos_in]) * 8
            mk = plsc.load_gather(tin.at[buf, 0], [pos_in])
            wk = plsc.load_gather(tin.at[buf, 1], [pos_in]) + T1
            m_idx = (mk >> 3) * (CP * 8) + it8 + (mk & 7)
            w_idx = (wk >> 3) * (CP * 8) + it8 + (wk & 7)
            pos = lane * 2 + j * (2 * LANES)
            plsc.store_scatter(idx2.at[buf], [pos], m_idx)
            plsc.store_scatter(idx2.at[buf], [pos + 1], w_idx)
            return carry2

        lax.fori_loop(0, CHUNK // LANES, vec_body, 0)

    for h in in_copies(0, 0):
        h.start()

    def loop_body(i, carry):
        for par in range(2):
            k = 2 * i + par
            buf = par
            nbuf = 1 - par

            @pl.when(k < NCHUNK)
            def _():
                @pl.when(k + 1 < NCHUNK)
                def _():
                    for h in in_copies(k + 1, nbuf):
                        h.start()

                for h in in_copies(k, buf):
                    h.wait()

                @pl.when(k >= 2)
                def _():
                    out_copy(k - 2, buf).wait()

                build_idx(buf)
                gather_copy(buf).start()

                @pl.when(k >= 1)
                def _():
                    gather_copy(nbuf).wait()
                    out_copy(k - 1, nbuf).start()

        return carry

    lax.fori_loop(0, (NCHUNK + 1) // 2, loop_body, 0)
    last = NCHUNK - 1
    gather_copy(last & 1).wait()
    out_copy(last, last & 1).start()
    out_copy(last - 1, 1 - (last & 1)).wait()
    out_copy(last, last & 1).wait()


def _tc_transpose_body(m_ref, w_ref, o_ref):
    o_ref[0] = m_ref[0:128, :].T
    o_ref[1] = jnp.concatenate([m_ref[128:192, :], w_ref[0:64, :]], axis=0).T
    o_ref[2, :, 0:16] = w_ref[64:80, :].T


_tc_transpose = pl.pallas_call(
    _tc_transpose_body,
    grid=(_TGRID,),
    in_specs=[
        pl.BlockSpec((192, _TW), lambda p: (0, p)),
        pl.BlockSpec((80, _TW), lambda p: (0, p)),
    ],
    out_specs=pl.BlockSpec((3, _TW, 128), lambda p: (0, p, 0)),
    out_shape=jax.ShapeDtypeStruct((3, CP, 128), jnp.float32),
)


def _tc_relayout_body(x_ref, o_ref):
    for i in range(8):
        z = x_ref[i].T
        for g in range(4):
            o_ref[i, :, pl.ds(g * 1024, 1024)] = z[32 * g:32 * (g + 1), :]


_tc_relayout = pl.pallas_call(
    _tc_relayout_body,
    grid=(L // 8,),
    in_specs=[pl.BlockSpec((8, 1024, 128), lambda l: (l, 0, 0))],
    out_specs=pl.BlockSpec((8, 2 * BASE, B), lambda l: (l, 0, 0)),
    out_shape=jax.ShapeDtypeStruct((L, 2 * BASE, B), jnp.float32),
)


def kernel(log_seqs, time1_seqs, time2_seqs, month_pop_table, week_pop_table):
    table = _tc_transpose(month_pop_table, week_pop_table).reshape(3 * CP * 8, BASE)
    t1 = time1_seqs.T.reshape(-1).astype(jnp.int32)
    t2 = time2_seqs.T.reshape(-1).astype(jnp.int32)
    item = log_seqs.T.reshape(-1).astype(jnp.int32)
    rows = _sc_gather(table, t1, t2, item)
    planes = _tc_relayout(rows.reshape(L, 1024, 128))
    return planes.transpose(2, 0, 1)
